# baseline (speedup 1.0000x reference)
import functools

import jax
import jax.numpy as jnp
from jax import lax
from jax.experimental import pallas as pl
from jax.experimental.pallas import tpu as pltpu
from jax.experimental.pallas import tpu_sc as plsc

N_ROWS = 100000
RAW_COLS = 64
N_COL = 128
BATCH = 16384

_info = plsc.get_sparse_core_info()
NC = _info.num_cores
NS = _info.num_subcores
L = _info.num_lanes
NW = NC * NS
BPW = BATCH // NW
G = BPW // L
CHUNK = 128
NCHUNK = BPW // CHUNK

_mesh = plsc.VectorSubcoreMesh(core_axis_name="c", subcore_axis_name="s")

@functools.partial(
    pl.kernel,
    mesh=_mesh,
    compiler_params=pltpu.CompilerParams(
        use_tc_tiling_on_sc=False, needs_layout_passes=False
    ),
    out_type=jax.ShapeDtypeStruct((BATCH, N_COL), jnp.float32),
    scratch_types=[
        pltpu.VMEM((BPW,), jnp.int32),
        pltpu.VMEM((BPW, RAW_COLS), jnp.float32),
        pltpu.VMEM((BPW, RAW_COLS), jnp.float32),
        pltpu.SemaphoreType.DMA,
    ],
)
def _lookup(table_hbm, idx_hbm, out_hbm, idx_v, rows_v, pad_v, sem):
    wid = lax.axis_index("s") * NC + lax.axis_index("c")
    base = wid * BPW
    iota = lax.iota(jnp.int32, L)

    pltpu.sync_copy(idx_hbm.at[pl.ds(base, BPW)], idx_v)

    copies = []
    for j in range(NCHUNK):
        copies.append(
            pltpu.async_copy(
                table_hbm.at[idx_v.at[pl.ds(j * CHUNK, CHUNK)]],
                rows_v.at[pl.ds(j * CHUNK, CHUNK)],
                sem,
            )
        )
    for c in copies:
        c.wait()

    def fix(g, carry):
        rowidx = g * L + iota
        last = plsc.load_gather(
            rows_v, [rowidx, jnp.full((L,), RAW_COLS - 1, jnp.int32)]
        )
        for c in range(RAW_COLS):
            col = c + iota
            col = jnp.where(col >= RAW_COLS, col - RAW_COLS, col)
            plsc.store_scatter(pad_v, [rowidx, col], last)
        return carry

    lax.fori_loop(0, G, fix, 0)

    pltpu.sync_copy(
        rows_v, out_hbm.at[pl.ds(base, BPW), pl.ds(0, RAW_COLS)]
    )
    pltpu.sync_copy(
        pad_v, out_hbm.at[pl.ds(base, BPW), pl.ds(RAW_COLS, RAW_COLS)]
    )


def kernel(table, index):
    return _lookup(table, index)


# speedup vs baseline: 1.4582x; 1.4582x over previous
import functools

import jax
import jax.numpy as jnp
from jax import lax
from jax.experimental import pallas as pl
from jax.experimental.pallas import tpu as pltpu
from jax.experimental.pallas import tpu_sc as plsc

N_ROWS = 100000
RAW_COLS = 64
N_COL = 128
BATCH = 16384

_info = plsc.get_sparse_core_info()
NC = _info.num_cores
NS = _info.num_subcores
L = _info.num_lanes
NW = NC * NS
BPW = BATCH // NW
G = BPW // L

_mesh = plsc.VectorSubcoreMesh(core_axis_name="c", subcore_axis_name="s")

@functools.partial(
    pl.kernel,
    mesh=_mesh,
    compiler_params=pltpu.CompilerParams(needs_layout_passes=False),
    out_type=jax.ShapeDtypeStruct((BATCH, N_COL), jnp.float32),
    scratch_types=[
        pltpu.VMEM((BPW,), jnp.int32),
        pltpu.VMEM((BPW, N_COL), jnp.float32),
        pltpu.VMEM((BPW // 2, N_COL), jnp.float32),
        pltpu.SemaphoreType.DMA,
        pltpu.SemaphoreType.DMA,
    ],
)
def _lookup(table_hbm, idx_hbm, out_hbm, idx_v, rows_v, drain_v, sem, sem2):
    wid = lax.axis_index("s") * NC + lax.axis_index("c")
    base = wid * BPW
    iota = lax.iota(jnp.int32, L)

    pltpu.sync_copy(idx_hbm.at[pl.ds(base, BPW)], idx_v)

    def issue(g, carry):
        v = idx_v[pl.ds(g * L, L)]
        for k in range(L):
            pltpu.async_copy(
                table_hbm.at[v[k]],
                rows_v.at[g * L + k, pl.ds(0, RAW_COLS)],
                sem,
            )
        return carry

    lax.fori_loop(0, G, issue, 0)

    pltpu.make_async_copy(
        out_hbm.at[pl.ds(0, BPW // 2)], drain_v, sem
    ).wait()

    def fix(g, carry):
        rowidx = g * L + iota
        last = plsc.load_gather(
            rows_v, [rowidx, jnp.full((L,), RAW_COLS - 1, jnp.int32)]
        )
        for c in range(RAW_COLS):
            col = c + iota
            col = jnp.where(col >= RAW_COLS, col - RAW_COLS, col)
            plsc.store_scatter(rows_v, [rowidx, col + RAW_COLS], last)
        return carry

    lax.fori_loop(0, G, fix, 0)

    pltpu.async_copy(rows_v, out_hbm.at[pl.ds(base, BPW)], sem2).wait()


def kernel(table, index):
    return _lookup(table, index)
